# baseline (speedup 1.0000x reference)
import functools

import jax
import jax.numpy as jnp
from jax import lax
from jax.experimental import pallas as pl
from jax.experimental.pallas import tpu as pltpu
from jax.experimental.pallas import tpu_sc as plsc

NUM_CORES = 2
NUM_SUBCORES = 16
NUM_WORKERS = NUM_CORES * NUM_SUBCORES

S_CHUNK = 2
NBUF = 4
LANE = 16
BBLK = 128
TPAD = BBLK + 1


def _permute_idx_tc(idx_t):
    s, b = idx_t.shape
    n_tb = b // BBLK
    per_w = s * BBLK

    def body(x_ref, o_ref):
        o_ref[...] = x_ref[...].reshape(per_w)

    return pl.pallas_call(
        body,
        grid=(n_tb,),
        in_specs=[pl.BlockSpec((s, BBLK), lambda tb: (0, tb))],
        out_specs=pl.BlockSpec((per_w,), lambda tb: (tb,)),
        out_shape=jax.ShapeDtypeStruct((b * s,), jnp.int32),
    )(idx_t)


@functools.partial(jax.jit, static_argnums=(3, 4, 5))
def _pos_embed_sc(perm_idx, token_table, position_table, batch, seq_len, dim):
    n = perm_idx.shape[0]
    per_w = n // NUM_WORKERS
    chunk = S_CHUNK * BBLK
    n_chunks = seq_len // S_CHUNK
    n_tc = dim // 8
    n_tb = batch // BBLK
    trows = S_CHUNK * n_tc * 8
    assert n_chunks % NBUF == 0 and per_w == n_chunks * chunk

    mesh = plsc.VectorSubcoreMesh(
        core_axis_name="c", subcore_axis_name="s",
        num_cores=NUM_CORES, num_subcores=NUM_SUBCORES)

    @functools.partial(
        pl.kernel,
        mesh=mesh,
        compiler_params=pltpu.CompilerParams(
            use_tc_tiling_on_sc=False, needs_layout_passes=False),
        out_type=jax.ShapeDtypeStruct(
            (seq_len, n_tc, n_tb, 8, BBLK), jnp.float32),
        scratch_types=(
            [pltpu.VMEM((chunk,), jnp.int32) for _ in range(NBUF)]
            + [pltpu.VMEM((chunk, dim), jnp.float32) for _ in range(NBUF)]
            + [pltpu.VMEM((trows, TPAD), jnp.float32) for _ in range(NBUF)]
            + [pltpu.VMEM((seq_len, dim), jnp.float32)]
            + [pltpu.SemaphoreType.DMA for _ in range(3 * NBUF)]
        ),
    )
    def sc_kernel(idx_hbm, tok_hbm, pos_hbm, out_hbm, *scratch):
        idx_v = scratch[:NBUF]
        gbuf = scratch[NBUF:2 * NBUF]
        tbuf = scratch[2 * NBUF:3 * NBUF]
        pos_v = scratch[3 * NBUF]
        sems = scratch[3 * NBUF + 1:]
        sem_i = sems[:NBUF]
        sem_g = sems[NBUF:2 * NBUF]
        sem_s = sems[2 * NBUF:]

        wid = lax.axis_index("s") * NUM_CORES + lax.axis_index("c")
        tok_base = wid * per_w

        pltpu.sync_copy(pos_hbm, pos_v)
        ci = lax.iota(jnp.int32, LANE)
        rowp = (ci >> 3) * 8 + (ci & 7)

        def issue_idx(c, b):
            off = tok_base + c * chunk
            pltpu.async_copy(idx_hbm.at[pl.ds(off, chunk)], idx_v[b], sem_i[b])

        def wait_idx(b):
            pltpu.make_async_copy(
                idx_hbm.at[pl.ds(0, chunk)], idx_v[b], sem_i[b]).wait()

        def issue_gather(b):
            pltpu.async_copy(tok_hbm.at[idx_v[b]], gbuf[b], sem_g[b])

        def wait_gather(b):
            pltpu.make_async_copy(tok_hbm.at[idx_v[b]], gbuf[b], sem_g[b]).wait()

        def issue_scatter(c, b):
            s0 = c * S_CHUNK
            for sl in range(S_CHUNK):
                for tc in range(n_tc):
                    pltpu.async_copy(
                        tbuf[b].at[pl.ds((sl * n_tc + tc) * 8, 8), pl.ds(0, BBLK)],
                        out_hbm.at[s0 + sl, tc, wid], sem_s[b])

        def wait_scatter(b):
            for _ in range(S_CHUNK * n_tc):
                pltpu.make_async_copy(
                    tbuf[b].at[pl.ds(0, 8), pl.ds(0, BBLK)],
                    out_hbm.at[0, 0, 0], sem_s[b]).wait()

        def transpose_add(c, b):
            s0 = c * S_CHUNK
            for sl in range(S_CHUNK):
                p0 = pos_v[s0 + sl, pl.ds(0, LANE)]
                p1 = pos_v[s0 + sl, pl.ds(LANE, LANE)]
                row0 = rowp + sl * (n_tc * 8)
                row1 = row0 + 2 * 8

                def bl_body(bl):
                    j = sl * BBLK + bl
                    col = jnp.full((LANE,), bl, jnp.int32)
                    v0 = gbuf[b][j, pl.ds(0, LANE)] + p0
                    v1 = gbuf[b][j, pl.ds(LANE, LANE)] + p1
                    plsc.store_scatter(tbuf[b], [row0, col], v0)
                    plsc.store_scatter(tbuf[b], [row1, col], v1)

                plsc.parallel_loop(0, BBLK, 1, unroll=4)(bl_body)

        for k in range(NBUF - 1):
            issue_idx(k, k)
        for k in range(NBUF - 2):
            wait_idx(k)
            issue_gather(k)

        def outer(i, _):
            for b in range(NBUF):
                c = i * NBUF + b
                gi = c + NBUF - 1
                gg = c + NBUF - 2
                bi = (b + NBUF - 1) % NBUF
                bg = (b + NBUF - 2) % NBUF

                @pl.when(gi < n_chunks)
                def _issue_idx():
                    issue_idx(gi, bi)

                @pl.when(gg < n_chunks)
                def _issue_gather():
                    wait_idx(bg)
                    issue_gather(bg)

                wait_gather(b)

                @pl.when(c >= NBUF)
                def _drain():
                    wait_scatter(b)

                transpose_add(c, b)
                issue_scatter(c, b)
            return ()

        lax.fori_loop(0, n_chunks // NBUF, outer, ())

        for b in range(NBUF):
            wait_scatter(b)

    return sc_kernel(perm_idx, token_table, position_table)


def kernel(inputs, token_table, position_table):
    b, s = inputs.shape
    dim = token_table.shape[1]
    perm_idx = _permute_idx_tc(inputs.astype(jnp.int32).transpose(1, 0))
    raw = _pos_embed_sc(perm_idx, token_table, position_table, b, s, dim)
    return raw.transpose(2, 4, 0, 1, 3).reshape(b, s, dim)


# speedup vs baseline: 1.0676x; 1.0676x over previous
import functools

import jax
import jax.numpy as jnp
from jax import lax
from jax.experimental import pallas as pl
from jax.experimental.pallas import tpu as pltpu
from jax.experimental.pallas import tpu_sc as plsc

NUM_CORES = 2
NUM_SUBCORES = 16
NUM_WORKERS = NUM_CORES * NUM_SUBCORES

S_CHUNK = 2
NBUF = 5
LANE = 16
BBLK = 128
TPAD = BBLK + 1


@functools.partial(jax.jit, static_argnums=(3, 4, 5))
def _pos_embed_sc(perm_idx, token_table, position_table, batch, seq_len, dim):
    n = perm_idx.shape[0]
    per_w = n // NUM_WORKERS
    chunk = S_CHUNK * BBLK
    n_chunks = seq_len // S_CHUNK
    n_tc = dim // 8
    n_tb = batch // BBLK
    trows = S_CHUNK * n_tc * 8
    assert n_chunks % NBUF == 0 and per_w == n_chunks * chunk

    mesh = plsc.VectorSubcoreMesh(
        core_axis_name="c", subcore_axis_name="s",
        num_cores=NUM_CORES, num_subcores=NUM_SUBCORES)

    @functools.partial(
        pl.kernel,
        mesh=mesh,
        compiler_params=pltpu.CompilerParams(
            use_tc_tiling_on_sc=False, needs_layout_passes=False),
        out_type=jax.ShapeDtypeStruct(
            (seq_len, n_tc, n_tb, 8, BBLK), jnp.float32),
        scratch_types=(
            [pltpu.VMEM((chunk,), jnp.int32) for _ in range(NBUF)]
            + [pltpu.VMEM((chunk, dim), jnp.float32) for _ in range(NBUF)]
            + [pltpu.VMEM((trows, TPAD), jnp.float32) for _ in range(NBUF)]
            + [pltpu.VMEM((seq_len, dim), jnp.float32)]
            + [pltpu.SemaphoreType.DMA for _ in range(3 * NBUF)]
        ),
    )
    def sc_kernel(idx_hbm, tok_hbm, pos_hbm, out_hbm, *scratch):
        idx_v = scratch[:NBUF]
        gbuf = scratch[NBUF:2 * NBUF]
        tbuf = scratch[2 * NBUF:3 * NBUF]
        pos_v = scratch[3 * NBUF]
        sems = scratch[3 * NBUF + 1:]
        sem_i = sems[:NBUF]
        sem_g = sems[NBUF:2 * NBUF]
        sem_s = sems[2 * NBUF:]

        wid = lax.axis_index("s") * NUM_CORES + lax.axis_index("c")
        tok_base = wid * per_w

        pltpu.sync_copy(pos_hbm, pos_v)
        ci = lax.iota(jnp.int32, LANE)
        rowp = (ci >> 3) * 8 + (ci & 7)

        def issue_idx(c, b):
            off = tok_base + c * chunk
            pltpu.async_copy(idx_hbm.at[pl.ds(off, chunk)], idx_v[b], sem_i[b])

        def wait_idx(b):
            pltpu.make_async_copy(
                idx_hbm.at[pl.ds(0, chunk)], idx_v[b], sem_i[b]).wait()

        def issue_gather(b):
            pltpu.async_copy(tok_hbm.at[idx_v[b]], gbuf[b], sem_g[b])

        def wait_gather(b):
            pltpu.make_async_copy(tok_hbm.at[idx_v[b]], gbuf[b], sem_g[b]).wait()

        def issue_scatter(c, b):
            s0 = c * S_CHUNK
            for sl in range(S_CHUNK):
                for tc in range(n_tc):
                    pltpu.async_copy(
                        tbuf[b].at[pl.ds((sl * n_tc + tc) * 8, 8), pl.ds(0, BBLK)],
                        out_hbm.at[s0 + sl, tc, wid], sem_s[b])

        def wait_scatter(b):
            for _ in range(S_CHUNK * n_tc):
                pltpu.make_async_copy(
                    tbuf[b].at[pl.ds(0, 8), pl.ds(0, BBLK)],
                    out_hbm.at[0, 0, 0], sem_s[b]).wait()

        def transpose_add(c, b):
            s0 = c * S_CHUNK
            for sl in range(S_CHUNK):
                p0 = pos_v[s0 + sl, pl.ds(0, LANE)]
                p1 = pos_v[s0 + sl, pl.ds(LANE, LANE)]
                row0 = rowp + sl * (n_tc * 8)
                row1 = row0 + 2 * 8

                def bl_body(bl):
                    j = sl * BBLK + bl
                    col = jnp.full((LANE,), bl, jnp.int32)
                    v0 = gbuf[b][j, pl.ds(0, LANE)] + p0
                    v1 = gbuf[b][j, pl.ds(LANE, LANE)] + p1
                    plsc.store_scatter(tbuf[b], [row0, col], v0)
                    plsc.store_scatter(tbuf[b], [row1, col], v1)

                plsc.parallel_loop(0, BBLK, 1, unroll=4)(bl_body)

        for k in range(NBUF - 1):
            issue_idx(k, k)
        for k in range(NBUF - 2):
            wait_idx(k)
            issue_gather(k)

        def outer(i, _):
            for b in range(NBUF):
                c = i * NBUF + b
                gi = c + NBUF - 1
                gg = c + NBUF - 2
                bi = (b + NBUF - 1) % NBUF
                bg = (b + NBUF - 2) % NBUF

                @pl.when(gi < n_chunks)
                def _issue_idx():
                    issue_idx(gi, bi)

                @pl.when(gg < n_chunks)
                def _issue_gather():
                    wait_idx(bg)
                    issue_gather(bg)

                wait_gather(b)

                @pl.when(c >= NBUF)
                def _drain():
                    wait_scatter(b)

                transpose_add(c, b)
                issue_scatter(c, b)
            return ()

        lax.fori_loop(0, n_chunks // NBUF, outer, ())

        for b in range(NBUF):
            wait_scatter(b)

    return sc_kernel(perm_idx, token_table, position_table)


def kernel(inputs, token_table, position_table):
    b, s = inputs.shape
    dim = token_table.shape[1]
    rows_per_w = b // NUM_WORKERS
    perm_idx = (inputs.astype(jnp.int32)
                .reshape(NUM_WORKERS, rows_per_w, s)
                .transpose(0, 2, 1)
                .reshape(b * s))
    raw = _pos_embed_sc(perm_idx, token_table, position_table, b, s, dim)
    return raw.transpose(2, 4, 0, 1, 3).reshape(b, s, dim)
